# baseline (speedup 1.0000x reference)
import functools

import jax
import jax.numpy as jnp
from jax import lax
from jax.experimental import pallas as pl
from jax.experimental.pallas import tpu as pltpu
from jax.experimental.pallas import tpu_sc as plsc

NUM_ROWS = 1_000_000
DIM = 64
BATCH = 16384
HIST = 200
HALF = HIST // 2
PADH = 104
BUF = 2 * PADH
NVR = BUF // 16

NC = 2
NS = 16
NW = NC * NS
BPW = BATCH // NW
CHUNK = 256
NCHUNKS = BPW // CHUNK
NBUF = 16
NGROUPS = CHUNK // NBUF


RB = 8192
NFULL = NUM_ROWS // RB
TAIL = NUM_ROWS - NFULL * RB
NBLK = NFULL + 1
QTOT = NBLK * RB


def _tc_heads_packed(table, w12):
    def body(t_hbm, w_ref, o_ref, tb, sems):
        i = pl.program_id(0)

        def issue(j, slot):
            @pl.when(j < NFULL)
            def _():
                pltpu.async_copy(t_hbm.at[pl.ds(j * RB, RB)],
                                 tb.at[slot], sems.at[slot])

            @pl.when(j == NFULL)
            def _():
                pltpu.async_copy(t_hbm.at[pl.ds(NFULL * RB, TAIL)],
                                 tb.at[slot].at[pl.ds(0, TAIL)],
                                 sems.at[slot])

        def drain(j, slot):
            @pl.when(j < NFULL)
            def _():
                pltpu.make_async_copy(t_hbm.at[pl.ds(j * RB, RB)],
                                      tb.at[slot], sems.at[slot]).wait()

            @pl.when(j == NFULL)
            def _():
                pltpu.make_async_copy(t_hbm.at[pl.ds(NFULL * RB, TAIL)],
                                      tb.at[slot].at[pl.ds(0, TAIL)],
                                      sems.at[slot]).wait()

        def per_parity(fn, j):
            @pl.when(lax.rem(j, 2) == 0)
            def _():
                fn(j, 0)

            @pl.when(lax.rem(j, 2) == 1)
            def _():
                fn(j, 1)

        @pl.when(i == 0)
        def _():
            per_parity(issue, 0)

        @pl.when(i + 1 < NBLK)
        def _():
            per_parity(issue, i + 1)

        per_parity(drain, i)

        def compute(j, slot):
            t = tb[slot]
            r = lax.dot_general(w_ref[...], t, (((1,), (1,)), ((), ())),
                                preferred_element_type=jnp.float32)
            u0 = lax.bitcast_convert_type(r[0], jnp.uint32)
            u1 = lax.bitcast_convert_type(r[1], jnp.uint32)
            r0 = (u0 + jnp.uint32(0x7FFF) + ((u0 >> 16) & jnp.uint32(1))) \
                & jnp.uint32(0xFFFF0000)
            r1 = (u1 + jnp.uint32(0x7FFF) + ((u1 >> 16) & jnp.uint32(1))) \
                & jnp.uint32(0xFFFF0000)
            q = (r0 >> 16) | r1
            o_ref[...] = lax.bitcast_convert_type(q, jnp.float32)

        per_parity(compute, i)

    return pl.pallas_call(
        body,
        grid=(NBLK,),
        in_specs=[
            pl.BlockSpec(memory_space=pl.ANY),
            pl.BlockSpec((2, DIM), lambda i: (0, 0)),
        ],
        out_specs=pl.BlockSpec((RB,), lambda i: (i,)),
        out_shape=jax.ShapeDtypeStruct((QTOT,), jnp.float32),
        scratch_shapes=[
            pltpu.VMEM((2, RB, DIM), jnp.float32),
            pltpu.SemaphoreType.DMA((2,)),
        ],
    )(table, w12)


def _sc_gather_reduce(x3, q, bias_vec):
    mesh = plsc.VectorSubcoreMesh(core_axis_name="c", subcore_axis_name="s",
                                  num_cores=NC, num_subcores=NS)

    @functools.partial(
        pl.kernel,
        out_type=[jax.ShapeDtypeStruct((BATCH,), jnp.float32),
                  jax.ShapeDtypeStruct((BATCH,), jnp.float32)],
        mesh=mesh,
        scratch_types=[
            pltpu.VMEM((CHUNK, 2, HALF), jnp.int32),
            pltpu.VMEM((NBUF, BUF), jnp.float32),
            pltpu.VMEM((CHUNK,), jnp.float32),
            pltpu.VMEM((CHUNK,), jnp.float32),
            pltpu.VMEM((16,), jnp.float32),
            pltpu.SemaphoreType.DMA((NBUF,)),
        ],
        compiler_params=pltpu.CompilerParams(use_tc_tiling_on_sc=False,
                                             needs_layout_passes=False),
    )
    def body(x_hbm, q_hbm, bias_hbm, out1_hbm, out2_hbm,
             idx_v, buf_v, o1_v, o2_v, bias_v, sems):
        wid = lax.axis_index("s") * NC + lax.axis_index("c")
        base = wid * BPW
        pltpu.sync_copy(bias_hbm, bias_v)
        bv = bias_v[...]
        b1s = bv[0]
        b2s = bv[1]
        lanes = lax.iota(jnp.int32, 16)
        himask = jnp.broadcast_to(jnp.uint32(0xFFFF0000), (16,))

        zeros16 = jnp.broadcast_to(jnp.float32(0.0), (16,))
        for s in range(NBUF):
            for j in range(NVR):
                buf_v[s, pl.ds(16 * j, 16)] = zeros16

        def gathers(li, s):
            return [
                (q_hbm.at[idx_v.at[li, 0]], buf_v.at[s].at[pl.ds(0, HALF)]),
                (q_hbm.at[idx_v.at[li, 1]], buf_v.at[s].at[pl.ds(PADH, HALF)]),
            ]

        def issue(li, s):
            for src, dst in gathers(li, s):
                pltpu.async_copy(src, dst, sems.at[s])

        def drain(li, s):
            for src, dst in gathers(li, s):
                pltpu.make_async_copy(src, dst, sems.at[s]).wait()

        def reduce(s, v1, v2):
            acc1 = zeros16
            acc2 = zeros16
            for j in range(NVR):
                w = plsc.bitcast(buf_v[s, pl.ds(16 * j, 16)], jnp.uint32)
                acc1 = acc1 + plsc.bitcast(w << 16, jnp.float32)
                acc2 = acc2 + plsc.bitcast(w & himask, jnp.float32)
            s1 = jnp.sum(acc1) + b1s
            s2 = jnp.sum(acc2) + b2s
            sel = lanes == s
            v1 = jnp.where(sel, jnp.broadcast_to(s1, (16,)), v1)
            v2 = jnp.where(sel, jnp.broadcast_to(s2, (16,)), v2)
            return v1, v2

        def chunk_body(ci, _):
            cbase = base + ci * CHUNK
            pltpu.sync_copy(x_hbm.at[pl.ds(cbase, CHUNK)], idx_v)
            for b in range(NBUF):
                issue(b, b)

            def group_body(g, _):
                v1 = zeros16
                v2 = zeros16
                for b in range(NBUF):
                    li = g * NBUF + b
                    drain(li, b)
                    v1, v2 = reduce(b, v1, v2)

                    @pl.when(li + NBUF < CHUNK)
                    def _():
                        issue(li + NBUF, b)
                o1_v[pl.ds(g * NBUF, 16)] = v1
                o2_v[pl.ds(g * NBUF, 16)] = v2
                return 0

            lax.fori_loop(0, NGROUPS, group_body, 0)
            pltpu.sync_copy(o1_v, out1_hbm.at[pl.ds(cbase, CHUNK)])
            pltpu.sync_copy(o2_v, out2_hbm.at[pl.ds(cbase, CHUNK)])
            return 0

        lax.fori_loop(0, NCHUNKS, chunk_body, 0)

    return body(x3, q, bias_vec)


@jax.jit
def kernel(x, table, W1, b1, W2, b2):
    w12 = jnp.concatenate([W1, W2], axis=0) * (1.0 / HIST)
    bias_vec = jnp.concatenate(
        [b1, b2, jnp.zeros((14,), jnp.float32)])
    q = _tc_heads_packed(table, w12)
    x3 = x.astype(jnp.int32).reshape(BATCH, 2, HALF)
    out1, out2 = _sc_gather_reduce(x3, q, bias_vec)
    return (out1, out2)


# speedup vs baseline: 1.0466x; 1.0466x over previous
import functools

import jax
import jax.numpy as jnp
from jax import lax
from jax.experimental import pallas as pl
from jax.experimental.pallas import tpu as pltpu
from jax.experimental.pallas import tpu_sc as plsc

NUM_ROWS = 1_000_000
DIM = 64
BATCH = 16384
HIST = 200
HALF = HIST // 2
PADH = 104
BUF = 2 * PADH
NVR = BUF // 16

NC = 2
NS = 16
NW = NC * NS
BPW = BATCH // NW
CHUNK = 256
NCHUNKS = BPW // CHUNK
NBUF = 16
NGROUPS = CHUNK // NBUF


def _tc_heads_packed(table, w12):
    rb = 32768
    grid = pl.cdiv(NUM_ROWS, rb)

    def body(t_ref, w_ref, o_ref):
        r = lax.dot_general(w_ref[...], t_ref[...], (((1,), (1,)), ((), ())),
                            preferred_element_type=jnp.float32)
        u0 = lax.bitcast_convert_type(r[0], jnp.uint32)
        u1 = lax.bitcast_convert_type(r[1], jnp.uint32)
        r0 = (u0 + jnp.uint32(0x7FFF) + ((u0 >> 16) & jnp.uint32(1))) \
            & jnp.uint32(0xFFFF0000)
        r1 = (u1 + jnp.uint32(0x7FFF) + ((u1 >> 16) & jnp.uint32(1))) \
            & jnp.uint32(0xFFFF0000)
        q = (r0 >> 16) | r1
        o_ref[...] = lax.bitcast_convert_type(q, jnp.float32)

    return pl.pallas_call(
        body,
        grid=(grid,),
        in_specs=[
            pl.BlockSpec((rb, DIM), lambda i: (i, 0)),
            pl.BlockSpec((2, DIM), lambda i: (0, 0)),
        ],
        out_specs=pl.BlockSpec((rb,), lambda i: (i,)),
        out_shape=jax.ShapeDtypeStruct((NUM_ROWS,), jnp.float32),
    )(table, w12)


def _sc_gather_reduce(x3, q, bias_vec):
    mesh = plsc.VectorSubcoreMesh(core_axis_name="c", subcore_axis_name="s",
                                  num_cores=NC, num_subcores=NS)

    @functools.partial(
        pl.kernel,
        out_type=[jax.ShapeDtypeStruct((BATCH,), jnp.float32),
                  jax.ShapeDtypeStruct((BATCH,), jnp.float32)],
        mesh=mesh,
        scratch_types=[
            pltpu.VMEM((CHUNK, 2, HALF), jnp.int32),
            pltpu.VMEM((NBUF, BUF), jnp.float32),
            pltpu.VMEM((CHUNK,), jnp.float32),
            pltpu.VMEM((CHUNK,), jnp.float32),
            pltpu.VMEM((16,), jnp.float32),
            pltpu.SemaphoreType.DMA((NBUF,)),
        ],
        compiler_params=pltpu.CompilerParams(use_tc_tiling_on_sc=False,
                                             needs_layout_passes=False),
    )
    def body(x_hbm, q_hbm, bias_hbm, out1_hbm, out2_hbm,
             idx_v, buf_v, o1_v, o2_v, bias_v, sems):
        wid = lax.axis_index("s") * NC + lax.axis_index("c")
        base = wid * BPW
        pltpu.sync_copy(bias_hbm, bias_v)
        bv = bias_v[...]
        b1s = bv[0]
        b2s = bv[1]
        lanes = lax.iota(jnp.int32, 16)
        himask = jnp.broadcast_to(jnp.uint32(0xFFFF0000), (16,))

        zeros16 = jnp.broadcast_to(jnp.float32(0.0), (16,))
        for s in range(NBUF):
            for j in range(NVR):
                buf_v[s, pl.ds(16 * j, 16)] = zeros16

        def gathers(li, s):
            return [
                (q_hbm.at[idx_v.at[li, 0]], buf_v.at[s].at[pl.ds(0, HALF)]),
                (q_hbm.at[idx_v.at[li, 1]], buf_v.at[s].at[pl.ds(PADH, HALF)]),
            ]

        def issue(li, s):
            for src, dst in gathers(li, s):
                pltpu.async_copy(src, dst, sems.at[s])

        def drain(li, s):
            for src, dst in gathers(li, s):
                pltpu.make_async_copy(src, dst, sems.at[s]).wait()

        def reduce(s, v1, v2):
            acc1 = zeros16
            acc2 = zeros16
            for j in range(NVR):
                w = plsc.bitcast(buf_v[s, pl.ds(16 * j, 16)], jnp.uint32)
                acc1 = acc1 + plsc.bitcast(w << 16, jnp.float32)
                acc2 = acc2 + plsc.bitcast(w & himask, jnp.float32)
            s1 = jnp.sum(acc1) + b1s
            s2 = jnp.sum(acc2) + b2s
            sel = lanes == s
            v1 = jnp.where(sel, jnp.broadcast_to(s1, (16,)), v1)
            v2 = jnp.where(sel, jnp.broadcast_to(s2, (16,)), v2)
            return v1, v2

        def chunk_body(ci, _):
            cbase = base + ci * CHUNK
            pltpu.sync_copy(x_hbm.at[pl.ds(cbase, CHUNK)], idx_v)
            for b in range(NBUF):
                issue(b, b)

            def group_body(g, _):
                v1 = zeros16
                v2 = zeros16
                for b in range(NBUF):
                    li = g * NBUF + b
                    drain(li, b)
                    v1, v2 = reduce(b, v1, v2)

                    @pl.when(li + NBUF < CHUNK)
                    def _():
                        issue(li + NBUF, b)
                o1_v[pl.ds(g * NBUF, 16)] = v1
                o2_v[pl.ds(g * NBUF, 16)] = v2
                return 0

            lax.fori_loop(0, NGROUPS, group_body, 0)
            pltpu.sync_copy(o1_v, out1_hbm.at[pl.ds(cbase, CHUNK)])
            pltpu.sync_copy(o2_v, out2_hbm.at[pl.ds(cbase, CHUNK)])
            return 0

        lax.fori_loop(0, NCHUNKS, chunk_body, 0)

    return body(x3, q, bias_vec)


@jax.jit
def kernel(x, table, W1, b1, W2, b2):
    w12 = jnp.concatenate([W1, W2], axis=0) * (1.0 / HIST)
    bias_vec = jnp.concatenate(
        [b1, b2, jnp.zeros((14,), jnp.float32)])
    q = _tc_heads_packed(table, w12)
    x3 = x.astype(jnp.int32).reshape(BATCH, 2, HALF)
    out1, out2 = _sc_gather_reduce(x3, q, bias_vec)
    return (out1, out2)


# speedup vs baseline: 1.0950x; 1.0463x over previous
import functools

import jax
import jax.numpy as jnp
from jax import lax
from jax.experimental import pallas as pl
from jax.experimental.pallas import tpu as pltpu
from jax.experimental.pallas import tpu_sc as plsc

NUM_ROWS = 1_000_000
DIM = 64
BATCH = 16384
HIST = 200
HALF = HIST // 2
PADH = 104
BUF = 2 * PADH
NVR = BUF // 16

NC = 2
NS = 16
NW = NC * NS
BPW = BATCH // NW
CHUNK = 256
NCHUNKS = BPW // CHUNK
NBUF = 16
NGROUPS = CHUNK // NBUF


def _tc_heads_packed(table, w12):
    rb = 32768
    grid = pl.cdiv(NUM_ROWS, rb)

    def body(t_ref, w_ref, o_ref):
        r = lax.dot_general(w_ref[...], t_ref[...], (((1,), (1,)), ((), ())),
                            preferred_element_type=jnp.float32)
        u0 = lax.bitcast_convert_type(r[0], jnp.uint32)
        u1 = lax.bitcast_convert_type(r[1], jnp.uint32)
        r0 = (u0 + jnp.uint32(0x7FFF) + ((u0 >> 16) & jnp.uint32(1))) \
            & jnp.uint32(0xFFFF0000)
        r1 = (u1 + jnp.uint32(0x7FFF) + ((u1 >> 16) & jnp.uint32(1))) \
            & jnp.uint32(0xFFFF0000)
        q = (r0 >> 16) | r1
        o_ref[...] = lax.bitcast_convert_type(q, jnp.float32)

    return pl.pallas_call(
        body,
        grid=(grid,),
        in_specs=[
            pl.BlockSpec((rb, DIM), lambda i: (i, 0)),
            pl.BlockSpec((2, DIM), lambda i: (0, 0)),
        ],
        out_specs=pl.BlockSpec((rb,), lambda i: (i,)),
        out_shape=jax.ShapeDtypeStruct((NUM_ROWS,), jnp.float32),
    )(table, w12)


def _sc_gather_reduce(x3, q, bias_vec):
    mesh = plsc.VectorSubcoreMesh(core_axis_name="c", subcore_axis_name="s",
                                  num_cores=NC, num_subcores=NS)

    @functools.partial(
        pl.kernel,
        out_type=[jax.ShapeDtypeStruct((BATCH,), jnp.float32),
                  jax.ShapeDtypeStruct((BATCH,), jnp.float32)],
        mesh=mesh,
        scratch_types=[
            pltpu.VMEM((2 * CHUNK, HALF), jnp.int32),
            pltpu.VMEM((NBUF, BUF), jnp.float32),
            pltpu.VMEM((CHUNK,), jnp.float32),
            pltpu.VMEM((CHUNK,), jnp.float32),
            pltpu.VMEM((16,), jnp.float32),
            pltpu.SemaphoreType.DMA((NBUF,)),
        ],
        compiler_params=pltpu.CompilerParams(use_tc_tiling_on_sc=False,
                                             needs_layout_passes=False),
    )
    def body(x_hbm, q_hbm, bias_hbm, out1_hbm, out2_hbm,
             idx_v, buf_v, o1_v, o2_v, bias_v, sems):
        wid = lax.axis_index("s") * NC + lax.axis_index("c")
        base = wid * BPW
        pltpu.sync_copy(bias_hbm, bias_v)
        bv = bias_v[...]
        b1s = bv[0]
        b2s = bv[1]
        lanes = lax.iota(jnp.int32, 16)
        himask = jnp.broadcast_to(jnp.uint32(0xFFFF0000), (16,))

        zeros16 = jnp.broadcast_to(jnp.float32(0.0), (16,))
        for s in range(NBUF):
            for j in range(NVR):
                buf_v[s, pl.ds(16 * j, 16)] = zeros16

        def gathers(li, s):
            return [
                (q_hbm.at[idx_v.at[2 * li]], buf_v.at[s].at[pl.ds(0, HALF)]),
                (q_hbm.at[idx_v.at[2 * li + 1]], buf_v.at[s].at[pl.ds(PADH, HALF)]),
            ]

        def issue(li, s):
            for src, dst in gathers(li, s):
                pltpu.async_copy(src, dst, sems.at[s])

        def drain(li, s):
            for src, dst in gathers(li, s):
                pltpu.make_async_copy(src, dst, sems.at[s]).wait()

        def reduce(s, v1, v2):
            acc1 = zeros16
            acc2 = zeros16
            for j in range(NVR):
                w = plsc.bitcast(buf_v[s, pl.ds(16 * j, 16)], jnp.uint32)
                acc1 = acc1 + plsc.bitcast(w << 16, jnp.float32)
                acc2 = acc2 + plsc.bitcast(w & himask, jnp.float32)
            s1 = jnp.sum(acc1) + b1s
            s2 = jnp.sum(acc2) + b2s
            sel = lanes == s
            v1 = jnp.where(sel, jnp.broadcast_to(s1, (16,)), v1)
            v2 = jnp.where(sel, jnp.broadcast_to(s2, (16,)), v2)
            return v1, v2

        def chunk_body(ci, _):
            cbase = base + ci * CHUNK
            pltpu.sync_copy(x_hbm.at[pl.ds(2 * cbase, 2 * CHUNK)], idx_v)
            for b in range(NBUF):
                issue(b, b)

            def group_body(g, _):
                v1 = zeros16
                v2 = zeros16
                for b in range(NBUF):
                    li = g * NBUF + b
                    drain(li, b)
                    v1, v2 = reduce(b, v1, v2)

                    @pl.when(li + NBUF < CHUNK)
                    def _():
                        issue(li + NBUF, b)
                o1_v[pl.ds(g * NBUF, 16)] = v1
                o2_v[pl.ds(g * NBUF, 16)] = v2
                return 0

            lax.fori_loop(0, NGROUPS, group_body, 0)
            pltpu.sync_copy(o1_v, out1_hbm.at[pl.ds(cbase, CHUNK)])
            pltpu.sync_copy(o2_v, out2_hbm.at[pl.ds(cbase, CHUNK)])
            return 0

        lax.fori_loop(0, NCHUNKS, chunk_body, 0)

    return body(x3, q, bias_vec)


@jax.jit
def kernel(x, table, W1, b1, W2, b2):
    w12 = jnp.concatenate([W1, W2], axis=0) * (1.0 / HIST)
    bias_vec = jnp.concatenate(
        [b1, b2, jnp.zeros((14,), jnp.float32)])
    q = _tc_heads_packed(table, w12)
    x3 = x.astype(jnp.int32).reshape(2 * BATCH, HALF)
    out1, out2 = _sc_gather_reduce(x3, q, bias_vec)
    return (out1, out2)
